# baseline (speedup 1.0000x reference)
import dataclasses
import functools

import jax
import jax.numpy as jnp
from jax import lax
from jax.experimental import pallas as pl
from jax.experimental.pallas import tpu as pltpu
from jax.experimental.pallas import tpu_sc as plsc

B = 16384
M = 96
K = 256
DSUB = 8
D = M * DSUB

NC = 2
NS = 16
L = 16
NW = NC * NS
ROWS_PER_W = B // NW
RBLK = 8
NBLK = ROWS_PER_W // RBLK


def _sc_diff_partials(q, cp, cn, ptab):
    mesh = plsc.VectorSubcoreMesh(core_axis_name="c", subcore_axis_name="s")

    cparams = pltpu.CompilerParams()
    for _field, _val in (("needs_layout_passes", False),
                         ("use_tc_tiling_on_sc", False)):
        if _field in pltpu.CompilerParams.__dataclass_fields__:
            cparams = dataclasses.replace(cparams, **{_field: _val})

    @functools.partial(
        pl.kernel,
        out_type=jax.ShapeDtypeStruct((B, L), jnp.float32),
        mesh=mesh,
        compiler_params=cparams,
        scratch_types=[
            pltpu.VMEM((M * K * 4,), jnp.int32),
            pltpu.VMEM((2, RBLK, M), jnp.int32),
            pltpu.VMEM((2, RBLK, M), jnp.int32),
            pltpu.VMEM((2, RBLK, D // 2), jnp.int32),
            pltpu.VMEM((RBLK, L), jnp.float32),
            pltpu.SemaphoreType.DMA,
            pltpu.SemaphoreType.DMA,
            pltpu.SemaphoreType.DMA,
            pltpu.SemaphoreType.DMA,
            pltpu.SemaphoreType.DMA,
        ],
    )
    def sc_kernel(q_hbm, cp_hbm, cn_hbm, ptab_hbm, out_hbm,
                  tabv, cpv, cnv, qv, dacc,
                  sem_t, sem_c0, sem_c1, sem_q0, sem_q1):
        sems_c = (sem_c0, sem_c1)
        sems_q = (sem_q0, sem_q1)
        wid = lax.axis_index("c") * NS + lax.axis_index("s")
        base = wid * ROWS_PER_W

        tload = pltpu.async_copy(ptab_hbm, tabv, sem_t)

        lane = lax.iota(jnp.int32, L)
        quarter = lax.shift_right_logical(lane, 2)
        colpat = lax.bitwise_and(lane, 3)
        tpat = quarter * (K * 4) + colpat
        himask = jnp.int32(-65536)

        _dnums = lax.GatherDimensionNumbers(
            offset_dims=(), collapsed_slice_dims=(0,), start_index_map=(0,))

        def take16(x, idx):
            return lax.gather(x, idx[:, None], _dnums, (1,),
                              mode=lax.GatherScatterMode.PROMISE_IN_BOUNDS)

        def fire(blk, p):
            row0 = base + blk * RBLK
            pltpu.async_copy(cp_hbm.at[pl.ds(row0, RBLK)], cpv.at[p],
                             sems_c[p])
            pltpu.async_copy(cn_hbm.at[pl.ds(row0, RBLK)], cnv.at[p],
                             sems_c[p])
            pltpu.async_copy(q_hbm.at[pl.ds(row0, RBLK)], qv.at[p], sems_q[p])

        def drain(p):
            pltpu.make_async_copy(cp_hbm.at[pl.ds(0, RBLK)], cpv.at[p],
                                  sems_c[p]).wait()
            pltpu.make_async_copy(cn_hbm.at[pl.ds(0, RBLK)], cnv.at[p],
                                  sems_c[p]).wait()
            pltpu.make_async_copy(q_hbm.at[pl.ds(0, RBLK)], qv.at[p],
                                  sems_q[p]).wait()

        def compute(blk, p):
            row0 = base + blk * RBLK

            @pl.loop(0, RBLK, step=4)
            def _row(r0):
                rows = (r0, r0 + 1, r0 + 2, r0 + 3)
                acc_e = [jnp.zeros((L,), jnp.float32) for _ in rows]
                acc_o = [jnp.zeros((L,), jnp.float32) for _ in rows]
                for c in range(M // L):
                    ccp = [cpv.at[p][rr, pl.ds(c * L, L)] for rr in rows]
                    ccn = [cnv.at[p][rr, pl.ds(c * L, L)] for rr in rows]
                    for s in range(4):
                        g = 4 * c + s
                        take = quarter + 4 * s
                        goff = tpat + g * (4 * K * 4)
                        for i in range(4):
                            ep_i = lax.shift_left(take16(ccp[i], take),
                                                  2) + goff
                            en_i = lax.shift_left(take16(ccn[i], take),
                                                  2) + goff
                            tp = plsc.load_gather(tabv, [ep_i])
                            tn = plsc.load_gather(tabv, [en_i])
                            ep_e = plsc.bitcast(lax.shift_left(tp, 16),
                                                jnp.float32)
                            en_e = plsc.bitcast(lax.shift_left(tn, 16),
                                                jnp.float32)
                            ep_o = plsc.bitcast(
                                lax.bitwise_and(tp, himask), jnp.float32)
                            en_o = plsc.bitcast(
                                lax.bitwise_and(tn, himask), jnp.float32)
                            qw = qv.at[p][rows[i], pl.ds(g * L, L)]
                            qe = plsc.bitcast(lax.shift_left(qw, 16),
                                              jnp.float32)
                            qo = plsc.bitcast(
                                lax.bitwise_and(qw, himask), jnp.float32)
                            acc_e[i] = acc_e[i] + qe * (en_e - ep_e)
                            acc_o[i] = acc_o[i] + qo * (en_o - ep_o)
                for i in range(4):
                    dacc[rows[i], :] = acc_e[i] + acc_o[i]

            pltpu.sync_copy(dacc, out_hbm.at[pl.ds(row0, RBLK)])

        fire(0, 0)
        tload.wait()

        @pl.loop(0, NBLK, step=2)
        def _pair(blk0):
            for pp in (0, 1):
                blk = blk0 + pp

                @pl.when(blk + 1 < NBLK)
                def _prefetch():
                    fire(blk + 1, 1 - pp)

                drain(pp)
                compute(blk, pp)

    return sc_kernel(q, cp, cn, ptab)


def _pack_codebooks(codebooks):
    cb16 = codebooks.astype(jnp.bfloat16).reshape(M * K * 4, 2)
    return lax.bitcast_convert_type(cb16, jnp.int32)


def _pack_q(q):
    q16 = q.astype(jnp.bfloat16).reshape(B, D // 2, 2)
    return lax.bitcast_convert_type(q16, jnp.int32)


def _tc_loss(dparts):
    def body(x_ref, o_ref):
        d = jnp.sum(x_ref[...], axis=1)
        sp = jnp.maximum(d, 0.0) + jnp.log1p(jnp.exp(-jnp.abs(d)))
        o_ref[...] = jnp.reshape(jnp.sum(sp) * (1.0 / B), (1, 1))

    out = pl.pallas_call(
        body,
        out_shape=jax.ShapeDtypeStruct((1, 1), jnp.float32),
    )(dparts)
    return out[0, 0]


def kernel(q, pos_codes, neg_codes, codebooks):
    ptab = _pack_codebooks(codebooks)
    qp = _pack_q(q)
    cp = pos_codes.astype(jnp.int32)
    cn = neg_codes.astype(jnp.int32)
    dparts = _sc_diff_partials(qp, cp, cn, ptab)
    return _tc_loss(dparts)


# speedup vs baseline: 1.5894x; 1.5894x over previous
import dataclasses
import functools

import jax
import jax.numpy as jnp
from jax import lax
from jax.experimental import pallas as pl
from jax.experimental.pallas import tpu as pltpu
from jax.experimental.pallas import tpu_sc as plsc

B = 16384
M = 96
K = 256
DSUB = 8
D = M * DSUB

NC = 2
NS = 16
L = 16
NW = NC * NS
ROWS_PER_W = B // NW
RBLK = 8
NBLK = ROWS_PER_W // RBLK


def _sc_diff_partials(q, cp, cn, ptab):
    nb = q.shape[0]
    rows_per_w = nb // NW
    nblk = rows_per_w // RBLK
    mesh = plsc.VectorSubcoreMesh(core_axis_name="c", subcore_axis_name="s")

    cparams = pltpu.CompilerParams()
    for _field, _val in (("needs_layout_passes", False),
                         ("use_tc_tiling_on_sc", False)):
        if _field in pltpu.CompilerParams.__dataclass_fields__:
            cparams = dataclasses.replace(cparams, **{_field: _val})

    @functools.partial(
        pl.kernel,
        out_type=jax.ShapeDtypeStruct((nb, L), jnp.float32),
        mesh=mesh,
        compiler_params=cparams,
        scratch_types=[
            pltpu.VMEM((M * K * 4,), jnp.int32),
            pltpu.VMEM((2, RBLK, M), jnp.int32),
            pltpu.VMEM((2, RBLK, M), jnp.int32),
            pltpu.VMEM((2, RBLK, D), jnp.float32),
            pltpu.VMEM((RBLK, L), jnp.float32),
            pltpu.SemaphoreType.DMA,
            pltpu.SemaphoreType.DMA,
            pltpu.SemaphoreType.DMA,
            pltpu.SemaphoreType.DMA,
            pltpu.SemaphoreType.DMA,
        ],
    )
    def sc_kernel(q_hbm, cp_hbm, cn_hbm, ptab_hbm, out_hbm,
                  tabv, cpv, cnv, qv, dacc,
                  sem_t, sem_c0, sem_c1, sem_q0, sem_q1):
        sems_c = (sem_c0, sem_c1)
        sems_q = (sem_q0, sem_q1)
        wid = lax.axis_index("c") * NS + lax.axis_index("s")
        base = wid * rows_per_w

        tload = pltpu.async_copy(ptab_hbm, tabv, sem_t)

        lane = lax.iota(jnp.int32, L)
        quarter = lax.shift_right_logical(lane, 2)
        colpat = lax.bitwise_and(lane, 3)
        tpat = quarter * (K * 4) + colpat
        qe_pat = quarter * DSUB + colpat * 2
        qo_pat = qe_pat + 1
        himask = jnp.int32(-65536)

        _dnums = lax.GatherDimensionNumbers(
            offset_dims=(), collapsed_slice_dims=(0,), start_index_map=(0,))

        def take16(x, idx):
            return lax.gather(x, idx[:, None], _dnums, (1,),
                              mode=lax.GatherScatterMode.PROMISE_IN_BOUNDS)

        def fire(blk, p):
            row0 = base + blk * RBLK
            pltpu.async_copy(cp_hbm.at[pl.ds(row0, RBLK)], cpv.at[p],
                             sems_c[p])
            pltpu.async_copy(cn_hbm.at[pl.ds(row0, RBLK)], cnv.at[p],
                             sems_c[p])
            pltpu.async_copy(q_hbm.at[pl.ds(row0, RBLK)], qv.at[p], sems_q[p])

        def drain(p):
            pltpu.make_async_copy(cp_hbm.at[pl.ds(0, RBLK)], cpv.at[p],
                                  sems_c[p]).wait()
            pltpu.make_async_copy(cn_hbm.at[pl.ds(0, RBLK)], cnv.at[p],
                                  sems_c[p]).wait()
            pltpu.make_async_copy(q_hbm.at[pl.ds(0, RBLK)], qv.at[p],
                                  sems_q[p]).wait()

        def compute(blk, p):
            row0 = base + blk * RBLK

            @pl.loop(0, RBLK, step=4)
            def _row(r0):
                rows = (r0, r0 + 1, r0 + 2, r0 + 3)
                rsp = [jnp.full((L,), rr, jnp.int32) for rr in rows]
                acc_e = [jnp.zeros((L,), jnp.float32) for _ in rows]
                acc_o = [jnp.zeros((L,), jnp.float32) for _ in rows]
                for c in range(M // L):
                    ccp = [cpv.at[p][rr, pl.ds(c * L, L)] for rr in rows]
                    ccn = [cnv.at[p][rr, pl.ds(c * L, L)] for rr in rows]
                    for s in range(4):
                        g = 4 * c + s
                        take = quarter + 4 * s
                        goff = tpat + g * (4 * K * 4)
                        for i in range(4):
                            ep_i = lax.shift_left(take16(ccp[i], take),
                                                  2) + goff
                            en_i = lax.shift_left(take16(ccn[i], take),
                                                  2) + goff
                            tp = plsc.load_gather(tabv, [ep_i])
                            tn = plsc.load_gather(tabv, [en_i])
                            ep_e = plsc.bitcast(lax.shift_left(tp, 16),
                                                jnp.float32)
                            en_e = plsc.bitcast(lax.shift_left(tn, 16),
                                                jnp.float32)
                            ep_o = plsc.bitcast(
                                lax.bitwise_and(tp, himask), jnp.float32)
                            en_o = plsc.bitcast(
                                lax.bitwise_and(tn, himask), jnp.float32)
                            qe = plsc.load_gather(
                                qv.at[p], [rsp[i], qe_pat + g * 32])
                            qo = plsc.load_gather(
                                qv.at[p], [rsp[i], qo_pat + g * 32])
                            acc_e[i] = acc_e[i] + qe * (en_e - ep_e)
                            acc_o[i] = acc_o[i] + qo * (en_o - ep_o)
                for i in range(4):
                    dacc[rows[i], :] = acc_e[i] + acc_o[i]

            pltpu.sync_copy(dacc, out_hbm.at[pl.ds(row0, RBLK)])

        fire(0, 0)
        tload.wait()

        @pl.loop(0, nblk, step=2)
        def _pair(blk0):
            for pp in (0, 1):
                blk = blk0 + pp

                @pl.when(blk + 1 < nblk)
                def _prefetch():
                    fire(blk + 1, 1 - pp)

                drain(pp)
                compute(blk, pp)

    return sc_kernel(q, cp, cn, ptab)


def _pack_codebooks(codebooks):
    cb16 = codebooks.astype(jnp.bfloat16).reshape(M * K * 4, 2)
    return lax.bitcast_convert_type(cb16, jnp.int32)


def _tc_loss(d1, d2):
    def body(x1_ref, x2_ref, o_ref):
        tot = jnp.float32(0.0)
        for x_ref in (x1_ref, x2_ref):
            d = jnp.sum(x_ref[...], axis=1)
            sp = jnp.maximum(d, 0.0) + jnp.log1p(jnp.exp(-jnp.abs(d)))
            tot = tot + jnp.sum(sp)
        o_ref[...] = jnp.reshape(tot * (1.0 / B), (1, 1))

    out = pl.pallas_call(
        body,
        out_shape=jax.ShapeDtypeStruct((1, 1), jnp.float32),
    )(d1, d2)
    return out[0, 0]


def kernel(q, pos_codes, neg_codes, codebooks):
    ptab = _pack_codebooks(codebooks)
    cp = pos_codes.astype(jnp.int32)
    cn = neg_codes.astype(jnp.int32)
    h = B // 2
    d1 = _sc_diff_partials(q[:h], cp[:h], cn[:h], ptab)
    d2 = _sc_diff_partials(q[h:], cp[h:], cn[h:], ptab)
    return _tc_loss(d1, d2)


# speedup vs baseline: 1.8266x; 1.1492x over previous
import dataclasses
import functools

import jax
import jax.numpy as jnp
from jax import lax
from jax.experimental import pallas as pl
from jax.experimental.pallas import tpu as pltpu
from jax.experimental.pallas import tpu_sc as plsc

B = 16384
M = 96
K = 256
DSUB = 8
D = M * DSUB

NC = 2
NS = 16
L = 16
NW = NC * NS
ROWS_PER_W = B // NW
RBLK = 8
NBLK = ROWS_PER_W // RBLK


def _sc_diff_partials(q, cp, cn, ptab):
    mesh = plsc.VectorSubcoreMesh(core_axis_name="c", subcore_axis_name="s")

    cparams = pltpu.CompilerParams()
    for _field, _val in (("needs_layout_passes", False),
                         ("use_tc_tiling_on_sc", False)):
        if _field in pltpu.CompilerParams.__dataclass_fields__:
            cparams = dataclasses.replace(cparams, **{_field: _val})

    @functools.partial(
        pl.kernel,
        out_type=jax.ShapeDtypeStruct((B, L), jnp.float32),
        mesh=mesh,
        compiler_params=cparams,
        scratch_types=[
            pltpu.VMEM((M * K * 4,), jnp.int32),
            pltpu.VMEM((2, RBLK, M), jnp.int32),
            pltpu.VMEM((2, RBLK, M), jnp.int32),
            pltpu.VMEM((2, RBLK, D), jnp.float32),
            pltpu.VMEM((RBLK, L), jnp.float32),
            pltpu.SemaphoreType.DMA,
            pltpu.SemaphoreType.DMA,
            pltpu.SemaphoreType.DMA,
            pltpu.SemaphoreType.DMA,
            pltpu.SemaphoreType.DMA,
        ],
    )
    def sc_kernel(q_hbm, cp_hbm, cn_hbm, ptab_hbm, out_hbm,
                  tabv, cpv, cnv, qv, dacc,
                  sem_t, sem_c0, sem_c1, sem_q0, sem_q1):
        sems_c = (sem_c0, sem_c1)
        sems_q = (sem_q0, sem_q1)
        wid = lax.axis_index("c") * NS + lax.axis_index("s")
        base = wid * ROWS_PER_W

        tload = pltpu.async_copy(ptab_hbm, tabv, sem_t)

        lane = lax.iota(jnp.int32, L)
        quarter = lax.shift_right_logical(lane, 2)
        colpat = lax.bitwise_and(lane, 3)
        tpat = quarter * (K * 4) + colpat
        qe_pat = quarter * DSUB + colpat * 2
        qo_pat = qe_pat + 1
        himask = jnp.int32(-65536)

        _dnums = lax.GatherDimensionNumbers(
            offset_dims=(), collapsed_slice_dims=(0,), start_index_map=(0,))

        def take16(x, idx):
            return lax.gather(x, idx[:, None], _dnums, (1,),
                              mode=lax.GatherScatterMode.PROMISE_IN_BOUNDS)

        def fire(blk, p):
            row0 = base + blk * RBLK
            pltpu.async_copy(cp_hbm.at[pl.ds(row0, RBLK)], cpv.at[p],
                             sems_c[p])
            pltpu.async_copy(cn_hbm.at[pl.ds(row0, RBLK)], cnv.at[p],
                             sems_c[p])
            pltpu.async_copy(q_hbm.at[pl.ds(row0, RBLK)], qv.at[p], sems_q[p])

        def drain(p):
            pltpu.make_async_copy(cp_hbm.at[pl.ds(0, RBLK)], cpv.at[p],
                                  sems_c[p]).wait()
            pltpu.make_async_copy(cn_hbm.at[pl.ds(0, RBLK)], cnv.at[p],
                                  sems_c[p]).wait()
            pltpu.make_async_copy(q_hbm.at[pl.ds(0, RBLK)], qv.at[p],
                                  sems_q[p]).wait()

        def compute(blk, p):
            row0 = base + blk * RBLK

            @pl.loop(0, RBLK, step=4)
            def _row(r0):
                rows = (r0, r0 + 1, r0 + 2, r0 + 3)
                rsp = [jnp.full((L,), rr, jnp.int32) for rr in rows]
                acc_e = [jnp.zeros((L,), jnp.float32) for _ in rows]
                acc_o = [jnp.zeros((L,), jnp.float32) for _ in rows]
                for c in range(M // L):
                    ccp = [cpv.at[p][rr, pl.ds(c * L, L)] for rr in rows]
                    ccn = [cnv.at[p][rr, pl.ds(c * L, L)] for rr in rows]
                    for s in range(4):
                        g = 4 * c + s
                        take = quarter + 4 * s
                        goff = tpat + g * (4 * K * 4)
                        for i in range(4):
                            ep_i = lax.shift_left(take16(ccp[i], take),
                                                  2) + goff
                            en_i = lax.shift_left(take16(ccn[i], take),
                                                  2) + goff
                            tp = plsc.load_gather(tabv, [ep_i])
                            tn = plsc.load_gather(tabv, [en_i])
                            ep_e = plsc.bitcast(lax.shift_left(tp, 16),
                                                jnp.float32)
                            en_e = plsc.bitcast(lax.shift_left(tn, 16),
                                                jnp.float32)
                            ep_o = plsc.bitcast(
                                lax.bitwise_and(tp, himask), jnp.float32)
                            en_o = plsc.bitcast(
                                lax.bitwise_and(tn, himask), jnp.float32)
                            qe = plsc.load_gather(
                                qv.at[p], [rsp[i], qe_pat + g * 32])
                            qo = plsc.load_gather(
                                qv.at[p], [rsp[i], qo_pat + g * 32])
                            acc_e[i] = acc_e[i] + qe * (en_e - ep_e)
                            acc_o[i] = acc_o[i] + qo * (en_o - ep_o)
                for i in range(4):
                    dacc[rows[i], :] = acc_e[i] + acc_o[i]

            pltpu.sync_copy(dacc, out_hbm.at[pl.ds(row0, RBLK)])

        fire(0, 0)
        tload.wait()

        @pl.loop(0, NBLK, step=2)
        def _pair(blk0):
            for pp in (0, 1):
                blk = blk0 + pp

                @pl.when(blk + 1 < NBLK)
                def _prefetch():
                    fire(blk + 1, 1 - pp)

                drain(pp)
                compute(blk, pp)

    return sc_kernel(q, cp, cn, ptab)


def _pack_codebooks(codebooks):
    cb16 = codebooks.astype(jnp.bfloat16).reshape(M * K * 4, 2)
    return lax.bitcast_convert_type(cb16, jnp.int32)


def _tc_loss(dparts):
    def body(x_ref, o_ref):
        d = jnp.sum(x_ref[...], axis=1)
        sp = jnp.maximum(d, 0.0) + jnp.log1p(jnp.exp(-jnp.abs(d)))
        o_ref[...] = jnp.reshape(jnp.sum(sp) * (1.0 / B), (1, 1))

    out = pl.pallas_call(
        body,
        out_shape=jax.ShapeDtypeStruct((1, 1), jnp.float32),
    )(dparts)
    return out[0, 0]


def kernel(q, pos_codes, neg_codes, codebooks):
    ptab = _pack_codebooks(codebooks)
    cp = pos_codes.astype(jnp.int32)
    cn = neg_codes.astype(jnp.int32)
    dparts = _sc_diff_partials(q, cp, cn, ptab)
    return _tc_loss(dparts)
